# Initial kernel scaffold; baseline (speedup 1.0000x reference)
import jax
import jax.numpy as jnp
from jax.experimental import pallas as pl


def kernel(sequence, token_table, pos_table, gamma, beta):
    raise NotImplementedError("write your pallas kernel here")


# speedup vs baseline: 3.3073x; 3.3073x over previous
import jax
import jax.numpy as jnp
from jax.experimental import pallas as pl
from jax.experimental.pallas import tpu as pltpu
from jax.experimental.pallas import tpu_sc as plsc

EPS = 1e-5
GATHER_WINDOW = 128


def _sc_gather(token_table, flat_idx, n, d):
    mesh = plsc.VectorSubcoreMesh(core_axis_name="c", subcore_axis_name="s")

    @pl.kernel(
        out_type=jax.ShapeDtypeStruct((n, d), jnp.float32),
        mesh=mesh,
    )
    def gather_kernel(tab_hbm, idx_hbm, out_hbm):
        def body(idx_vmem, out_vmem):
            pltpu.sync_copy(tab_hbm.at[idx_vmem.at[0]], out_vmem)

        pltpu.emit_pipeline(
            body,
            grid=(n // GATHER_WINDOW,),
            in_specs=[
                pl.BlockSpec((1, GATHER_WINDOW), lambda i: (0, i)),
            ],
            out_specs=[
                pl.BlockSpec((GATHER_WINDOW, d), lambda i: (i, 0)),
            ],
            core_axis_name=("c", "s"),
            dimension_semantics=(pltpu.PARALLEL,),
        )(idx_hbm, out_hbm)

    return gather_kernel(token_table, flat_idx)


def _tc_norm_body(seq_ref, emb_ref, pos_ref, gam_ref, bet_ref, out_ref):
    x = emb_ref[...]
    m = (seq_ref[...] != 0).astype(jnp.float32)
    x = x * m + pos_ref[...]
    mu = jnp.mean(x, axis=-1, keepdims=True)
    xc = x - mu
    var = jnp.mean(xc * xc, axis=-1, keepdims=True)
    y = xc * jax.lax.rsqrt(var + EPS)
    out_ref[...] = y * gam_ref[...] + bet_ref[...]


def kernel(sequence, token_table, pos_table, gamma, beta):
    b, s = sequence.shape
    v, d = token_table.shape
    n = b * s

    flat_idx = sequence.reshape(1, n)
    gathered = _sc_gather(token_table, flat_idx, n, d)
    emb = gathered.reshape(b, s, d)

    bb = 8
    out = pl.pallas_call(
        _tc_norm_body,
        grid=(b // bb,),
        in_specs=[
            pl.BlockSpec((bb, s, 1), lambda i: (i, 0, 0)),
            pl.BlockSpec((bb, s, d), lambda i: (i, 0, 0)),
            pl.BlockSpec((1, s, d), lambda i: (0, 0, 0)),
            pl.BlockSpec((1, 1, d), lambda i: (0, 0, 0)),
            pl.BlockSpec((1, 1, d), lambda i: (0, 0, 0)),
        ],
        out_specs=pl.BlockSpec((bb, s, d), lambda i: (i, 0, 0)),
        out_shape=jax.ShapeDtypeStruct((b, s, d), jnp.float32),
    )(
        sequence.reshape(b, s, 1),
        emb,
        pos_table[:s].reshape(1, s, d),
        gamma.reshape(1, 1, d),
        beta.reshape(1, 1, d),
    )
    return out


# speedup vs baseline: 3.7137x; 1.1229x over previous
import jax
import jax.numpy as jnp
from jax.experimental import pallas as pl
from jax.experimental.pallas import tpu as pltpu
from jax.experimental.pallas import tpu_sc as plsc

EPS = 1e-5
GATHER_WINDOW = 128


def _sc_gather(token_table, flat_idx, n, d):
    mesh = plsc.VectorSubcoreMesh(core_axis_name="c", subcore_axis_name="s")

    @pl.kernel(
        out_type=jax.ShapeDtypeStruct((n, d), jnp.float32),
        mesh=mesh,
    )
    def gather_kernel(tab_hbm, idx_hbm, out_hbm):
        def body(idx_vmem, out_vmem):
            pltpu.sync_copy(tab_hbm.at[idx_vmem.at[0]], out_vmem)

        pltpu.emit_pipeline(
            body,
            grid=(n // GATHER_WINDOW,),
            in_specs=[
                pl.BlockSpec((1, GATHER_WINDOW), lambda i: (0, i)),
            ],
            out_specs=[
                pl.BlockSpec((GATHER_WINDOW, d), lambda i: (i, 0)),
            ],
            core_axis_name=("c", "s"),
            dimension_semantics=(pltpu.PARALLEL,),
        )(idx_hbm, out_hbm)

    return gather_kernel(token_table, flat_idx)


def _tc_norm_body(seq_ref, emb_ref, pos_ref, gam_ref, bet_ref, out_ref):
    x = emb_ref[...]
    m = (seq_ref[...] != 0).astype(jnp.float32)
    x = x * m + pos_ref[...]
    d = x.shape[-1]
    mu = jnp.sum(x, axis=-1, keepdims=True) * (1.0 / d)
    ex2 = jnp.sum(x * x, axis=-1, keepdims=True) * (1.0 / d)
    var = ex2 - mu * mu
    r = jax.lax.rsqrt(var + EPS)
    out_ref[...] = (x * r - mu * r) * gam_ref[...] + bet_ref[...]


def kernel(sequence, token_table, pos_table, gamma, beta):
    b, s = sequence.shape
    v, d = token_table.shape
    n = b * s

    flat_idx = sequence.reshape(1, n)
    gathered = _sc_gather(token_table, flat_idx, n, d)
    emb = gathered.reshape(b, s, d)

    bb = 16
    out = pl.pallas_call(
        _tc_norm_body,
        grid=(b // bb,),
        in_specs=[
            pl.BlockSpec((bb, s, 1), lambda i: (i, 0, 0)),
            pl.BlockSpec((bb, s, d), lambda i: (i, 0, 0)),
            pl.BlockSpec((1, s, d), lambda i: (0, 0, 0)),
            pl.BlockSpec((1, 1, d), lambda i: (0, 0, 0)),
            pl.BlockSpec((1, 1, d), lambda i: (0, 0, 0)),
        ],
        out_specs=pl.BlockSpec((bb, s, d), lambda i: (i, 0, 0)),
        out_shape=jax.ShapeDtypeStruct((b, s, d), jnp.float32),
    )(
        sequence.reshape(b, s, 1),
        emb,
        pos_table[:s].reshape(1, s, d),
        gamma.reshape(1, 1, d),
        beta.reshape(1, 1, d),
    )
    return out


# speedup vs baseline: 3.9869x; 1.0735x over previous
import jax
import jax.numpy as jnp
from jax.experimental import pallas as pl
from jax.experimental.pallas import tpu as pltpu
from jax.experimental.pallas import tpu_sc as plsc

EPS = 1e-5
GATHER_WINDOW = 128


def _sc_gather(token_table, flat_idx, n, d):
    mesh = plsc.VectorSubcoreMesh(core_axis_name="c", subcore_axis_name="s")

    @pl.kernel(
        out_type=jax.ShapeDtypeStruct((n, d), jnp.float32),
        mesh=mesh,
    )
    def gather_kernel(tab_hbm, idx_hbm, out_hbm):
        def body(idx_vmem, out_vmem):
            pltpu.sync_copy(tab_hbm.at[idx_vmem.at[0]], out_vmem)

        pltpu.emit_pipeline(
            body,
            grid=(n // GATHER_WINDOW,),
            in_specs=[
                pl.BlockSpec((1, GATHER_WINDOW), lambda i: (0, i)),
            ],
            out_specs=[
                pl.BlockSpec((GATHER_WINDOW, d), lambda i: (i, 0)),
            ],
            core_axis_name=("c", "s"),
            dimension_semantics=(pltpu.PARALLEL,),
        )(idx_hbm, out_hbm)

    return gather_kernel(token_table, flat_idx)


def _tc_norm_body(seq_ref, emb_ref, pos_ref, gam_ref, bet_ref, out_ref):
    x = emb_ref[...]
    m = (seq_ref[...] != 0).astype(jnp.float32)
    x = x * m + pos_ref[...]
    d = x.shape[-1]
    mu = jnp.sum(x, axis=-1, keepdims=True) * (1.0 / d)
    ex2 = jnp.sum(x * x, axis=-1, keepdims=True) * (1.0 / d)
    var = ex2 - mu * mu
    r = jax.lax.rsqrt(var + EPS)
    out_ref[...] = (x * r - mu * r) * gam_ref[...] + bet_ref[...]


def kernel(sequence, token_table, pos_table, gamma, beta):
    b, s = sequence.shape
    v, d = token_table.shape
    n = b * s

    flat_idx = sequence.reshape(1, n)
    gathered = _sc_gather(token_table, flat_idx, n, d)
    emb = gathered.reshape(b, s, d)

    bb = 32
    out = pl.pallas_call(
        _tc_norm_body,
        grid=(b // bb,),
        in_specs=[
            pl.BlockSpec((bb, s, 1), lambda i: (i, 0, 0)),
            pl.BlockSpec((bb, s, d), lambda i: (i, 0, 0)),
            pl.BlockSpec((1, s, d), lambda i: (0, 0, 0)),
            pl.BlockSpec((1, 1, d), lambda i: (0, 0, 0)),
            pl.BlockSpec((1, 1, d), lambda i: (0, 0, 0)),
        ],
        out_specs=pl.BlockSpec((bb, s, d), lambda i: (i, 0, 0)),
        out_shape=jax.ShapeDtypeStruct((b, s, d), jnp.float32),
    )(
        sequence.reshape(b, s, 1),
        emb,
        pos_table[:s].reshape(1, s, d),
        gamma.reshape(1, 1, d),
        beta.reshape(1, 1, d),
    )
    return out
